# baseline (speedup 1.0000x reference)
import functools

import jax
import jax.numpy as jnp
from jax import lax
from jax.experimental import pallas as pl
from jax.experimental.pallas import tpu as pltpu
from jax.experimental.pallas import tpu_sc as plsc

NUM_RELATIONS = 1000
EMBED_DIM = 64
HIDDEN = 128
B, S = 1024, 20
N = B * S
NC, NS = 2, 16
NW = NC * NS
PK = EMBED_DIM // 2
QROWS = N // 4
Q_PER_B = B // NW
Q_CHUNK = 1
CHUNKS_PER_SP = Q_PER_B // Q_CHUNK
N_CHUNKS = 5 * CHUNKS_PER_SP
IDX_PER_K = Q_CHUNK * S
NBUF = 2


def _sc_gather_mean(idx2d, table_u32):
    mesh = plsc.VectorSubcoreMesh(core_axis_name="c", subcore_axis_name="s")

    @functools.partial(
        pl.kernel,
        out_type=jax.ShapeDtypeStruct((QROWS, 4 * PK), jnp.uint32),
        mesh=mesh,
        scratch_types=[
            pltpu.VMEM((4, 5, Q_PER_B, S), jnp.int32),
            [pltpu.VMEM((4 * IDX_PER_K, PK), jnp.uint32)] * NBUF,
            [pltpu.VMEM((Q_CHUNK, 4 * PK), jnp.uint32)] * NBUF,
            [pltpu.SemaphoreType.DMA] * NBUF,
            [pltpu.SemaphoreType.DMA] * NBUF,
            pltpu.VMEM_SHARED((NUM_RELATIONS, PK), jnp.uint32),
        ],
        compiler_params=pltpu.CompilerParams(use_tc_tiling_on_sc=False),
    )
    def k(idx_hbm, table_hbm, agg_hbm, idx_v, rows_v, out_v, gsem, osem,
          tab_sh):
        wid = lax.axis_index("s") * NC + lax.axis_index("c")
        b0 = wid * Q_PER_B

        @pl.when(lax.axis_index("s") == 0)
        def _():
            pltpu.sync_copy(table_hbm, tab_sh)

        for kk in range(4):
            for sp in range(5):
                pltpu.sync_copy(
                    idx_hbm.at[pl.ds(b0, Q_PER_B), 5 * kk + sp],
                    idx_v.at[kk, sp],
                )
        plsc.subcore_barrier()

        def gather(t, b):
            sp = t // CHUNKS_PER_SP
            u = (t % CHUNKS_PER_SP) * Q_CHUNK
            for kk in range(4):
                pltpu.make_async_copy(
                    tab_sh.at[idx_v.at[kk, sp, u]],
                    rows_v[b].at[pl.ds(kk * IDX_PER_K, IDX_PER_K)],
                    gsem[b],
                ).start()

        def gather_wait(b):
            for kk in range(4):
                pltpu.make_async_copy(
                    tab_sh.at[idx_v.at[0, 0, 0]],
                    rows_v[b].at[pl.ds(kk * IDX_PER_K, IDX_PER_K)],
                    gsem[b],
                ).wait()

        def out_start(t, b):
            sp = t // CHUNKS_PER_SP
            u = (t % CHUNKS_PER_SP) * Q_CHUNK
            pltpu.make_async_copy(
                out_v[b],
                agg_hbm.at[pl.ds(sp * B + b0 + u, Q_CHUNK)],
                osem[b],
            ).start()

        def out_wait(b):
            pltpu.make_async_copy(
                out_v[b], agg_hbm.at[pl.ds(0, Q_CHUNK)], osem[b]
            ).wait()

        for b in range(NBUF):
            gather(b, b)

        def outer(tt, carry):
            for b in range(NBUF):
                t = tt * NBUF + b
                gather_wait(b)
                @pl.when(t >= NBUF)
                def _():
                    out_wait(b)

                rv, ov = rows_v[b], out_v[b]
                hi_mask = jnp.uint32(0xFFFF0000)
                for p in range(Q_CHUNK):
                    for kk in range(4):
                        for c in range(PK // 16):
                            acc_a = jnp.zeros((16,), jnp.float32)
                            acc_b = jnp.zeros((16,), jnp.float32)
                            for j in range(S):
                                u = rv[kk * IDX_PER_K + p * S + j,
                                       pl.ds(c * 16, 16)]
                                a = lax.bitcast_convert_type(
                                    u << 16, jnp.float32)
                                bb = lax.bitcast_convert_type(
                                    u & hi_mask, jnp.float32)
                                acc_a = acc_a + a
                                acc_b = acc_b + bb
                            ua = lax.bitcast_convert_type(
                                acc_a * (1.0 / S), jnp.uint32)
                            ub = lax.bitcast_convert_type(
                                acc_b * (1.0 / S), jnp.uint32)
                            ov[p, pl.ds(kk * PK + c * 16, 16)] = (
                                (ua >> 16) | (ub & hi_mask)
                            )
                out_start(t, b)

                @pl.when(t + NBUF < N_CHUNKS)
                def _():
                    gather(t + NBUF, b)

            return carry

        lax.fori_loop(0, N_CHUNKS // NBUF, outer, None)
        for b in range(NBUF):
            out_wait(b)

    return k(idx2d, table_u32)


GRID_I = 4
U_BLK = QROWS // GRID_I
BATCH_BLK = U_BLK // S


def _mlp(agg_u32, W1e, W1o, b1, W2, b2):
    def body(a_ref, w1e_ref, w1o_ref, b1_ref, w2_ref, b2_ref, o_ref):
        u = a_ref[...]
        a_even = lax.bitcast_convert_type(u << 16, jnp.float32)
        a_odd = lax.bitcast_convert_type(u & jnp.uint32(0xFFFF0000),
                                         jnp.float32)
        for kk in range(4):
            ek = a_even[:, kk * PK:(kk + 1) * PK]
            ok = a_odd[:, kk * PK:(kk + 1) * PK]
            h = (
                jnp.dot(ek, w1e_ref[...], preferred_element_type=jnp.float32)
                + jnp.dot(ok, w1o_ref[...],
                          preferred_element_type=jnp.float32)
            )
            h = jnp.maximum(h + b1_ref[...], 0.0)
            o = (
                jnp.dot(h, w2_ref[...], preferred_element_type=jnp.float32)
                + b2_ref[...]
            )
            o_ref[kk, 0] = o

    return pl.pallas_call(
        body,
        grid=(GRID_I,),
        in_specs=[
            pl.BlockSpec((U_BLK, 4 * PK), lambda i: (i, 0)),
            pl.BlockSpec((PK, HIDDEN), lambda i: (0, 0)),
            pl.BlockSpec((PK, HIDDEN), lambda i: (0, 0)),
            pl.BlockSpec((1, HIDDEN), lambda i: (0, 0)),
            pl.BlockSpec((HIDDEN, HIDDEN), lambda i: (0, 0)),
            pl.BlockSpec((1, HIDDEN), lambda i: (0, 0)),
        ],
        out_specs=pl.BlockSpec((4, 1, U_BLK, HIDDEN),
                               lambda i: (0, i, 0, 0)),
        out_shape=jax.ShapeDtypeStruct((4, GRID_I, U_BLK, HIDDEN),
                                       jnp.float32),
    )(agg_u32, W1e, W1o, b1, W2, b2)


def kernel(kg_spatial_matrix, rel_table, W1, b1, W2, b2):
    table_bf = rel_table.at[0].set(0.0).astype(jnp.bfloat16)
    table_u32 = jax.lax.bitcast_convert_type(
        table_bf.reshape(NUM_RELATIONS, PK, 2), jnp.uint32
    )
    agg_u32 = _sc_gather_mean(kg_spatial_matrix, table_u32)
    out4 = _mlp(agg_u32, W1[0::2], W1[1::2], b1.reshape(1, HIDDEN), W2,
                b2.reshape(1, HIDDEN))
    return out4.reshape(S, B, HIDDEN).transpose(1, 0, 2)


# speedup vs baseline: 1.0192x; 1.0192x over previous
import functools

import jax
import jax.numpy as jnp
from jax import lax
from jax.experimental import pallas as pl
from jax.experimental.pallas import tpu as pltpu
from jax.experimental.pallas import tpu_sc as plsc

NUM_RELATIONS = 1000
EMBED_DIM = 64
HIDDEN = 128
B, S = 1024, 20
N = B * S
NC, NS = 2, 16
NW = NC * NS
PK = EMBED_DIM // 2
QROWS = N // 4
Q_PER_B = B // NW
Q_CHUNK = 1
CHUNKS_PER_SP = Q_PER_B // Q_CHUNK
N_CHUNKS = 5 * CHUNKS_PER_SP
IDX_PER_K = Q_CHUNK * S
NBUF = 4


def _sc_gather_mean(idx2d, table_u32):
    mesh = plsc.VectorSubcoreMesh(core_axis_name="c", subcore_axis_name="s")

    @functools.partial(
        pl.kernel,
        out_type=jax.ShapeDtypeStruct((QROWS, 4 * PK), jnp.uint32),
        mesh=mesh,
        scratch_types=[
            pltpu.VMEM((4, 5, Q_PER_B, S), jnp.int32),
            [pltpu.VMEM((4 * IDX_PER_K, PK), jnp.uint32)] * NBUF,
            [pltpu.VMEM((Q_CHUNK, 4 * PK), jnp.uint32)] * NBUF,
            [pltpu.SemaphoreType.DMA] * NBUF,
            [pltpu.SemaphoreType.DMA] * NBUF,
            pltpu.VMEM_SHARED((NUM_RELATIONS, PK), jnp.uint32),
        ],
        compiler_params=pltpu.CompilerParams(use_tc_tiling_on_sc=False),
    )
    def k(idx_hbm, table_hbm, agg_hbm, idx_v, rows_v, out_v, gsem, osem,
          tab_sh):
        wid = lax.axis_index("s") * NC + lax.axis_index("c")
        b0 = wid * Q_PER_B

        @pl.when(lax.axis_index("s") == 0)
        def _():
            pltpu.sync_copy(table_hbm, tab_sh)

        for kk in range(4):
            for sp in range(5):
                pltpu.sync_copy(
                    idx_hbm.at[pl.ds(b0, Q_PER_B), 5 * kk + sp],
                    idx_v.at[kk, sp],
                )
        plsc.subcore_barrier()

        def gather(t, b):
            sp = t // CHUNKS_PER_SP
            u = (t % CHUNKS_PER_SP) * Q_CHUNK
            for kk in range(4):
                pltpu.make_async_copy(
                    tab_sh.at[idx_v.at[kk, sp, u]],
                    rows_v[b].at[pl.ds(kk * IDX_PER_K, IDX_PER_K)],
                    gsem[b],
                ).start()

        def gather_wait(b):
            for kk in range(4):
                pltpu.make_async_copy(
                    tab_sh.at[idx_v.at[0, 0, 0]],
                    rows_v[b].at[pl.ds(kk * IDX_PER_K, IDX_PER_K)],
                    gsem[b],
                ).wait()

        def out_start(t, b):
            sp = t // CHUNKS_PER_SP
            u = (t % CHUNKS_PER_SP) * Q_CHUNK
            pltpu.make_async_copy(
                out_v[b],
                agg_hbm.at[pl.ds(sp * B + b0 + u, Q_CHUNK)],
                osem[b],
            ).start()

        def out_wait(b):
            pltpu.make_async_copy(
                out_v[b], agg_hbm.at[pl.ds(0, Q_CHUNK)], osem[b]
            ).wait()

        for b in range(NBUF):
            gather(b, b)

        def outer(tt, carry):
            for b in range(NBUF):
                t = tt * NBUF + b
                gather_wait(b)
                @pl.when(t >= NBUF)
                def _():
                    out_wait(b)

                rv, ov = rows_v[b], out_v[b]
                hi_mask = jnp.uint32(0xFFFF0000)
                for p in range(Q_CHUNK):
                    for kk in range(4):
                        for c in range(PK // 16):
                            acc_a = jnp.zeros((16,), jnp.float32)
                            acc_b = jnp.zeros((16,), jnp.float32)
                            for j in range(S):
                                u = rv[kk * IDX_PER_K + p * S + j,
                                       pl.ds(c * 16, 16)]
                                a = lax.bitcast_convert_type(
                                    u << 16, jnp.float32)
                                bb = lax.bitcast_convert_type(
                                    u & hi_mask, jnp.float32)
                                acc_a = acc_a + a
                                acc_b = acc_b + bb
                            ua = lax.bitcast_convert_type(
                                acc_a * (1.0 / S), jnp.uint32)
                            ub = lax.bitcast_convert_type(
                                acc_b * (1.0 / S), jnp.uint32)
                            ov[p, pl.ds(kk * PK + c * 16, 16)] = (
                                (ua >> 16) | (ub & hi_mask)
                            )
                out_start(t, b)

                @pl.when(t + NBUF < N_CHUNKS)
                def _():
                    gather(t + NBUF, b)

            return carry

        lax.fori_loop(0, N_CHUNKS // NBUF, outer, None)
        for b in range(NBUF):
            out_wait(b)

    return k(idx2d, table_u32)


GRID_I = 4
U_BLK = QROWS // GRID_I
BATCH_BLK = U_BLK // S


def _mlp(agg_u32, W1e, W1o, b1, W2, b2):
    def body(a_ref, w1e_ref, w1o_ref, b1_ref, w2_ref, b2_ref, o_ref):
        u = a_ref[...]
        a_even = lax.bitcast_convert_type(u << 16, jnp.float32)
        a_odd = lax.bitcast_convert_type(u & jnp.uint32(0xFFFF0000),
                                         jnp.float32)
        for kk in range(4):
            ek = a_even[:, kk * PK:(kk + 1) * PK]
            ok = a_odd[:, kk * PK:(kk + 1) * PK]
            h = (
                jnp.dot(ek, w1e_ref[...], preferred_element_type=jnp.float32)
                + jnp.dot(ok, w1o_ref[...],
                          preferred_element_type=jnp.float32)
            )
            h = jnp.maximum(h + b1_ref[...], 0.0)
            o = (
                jnp.dot(h, w2_ref[...], preferred_element_type=jnp.float32)
                + b2_ref[...]
            )
            o_ref[kk, 0] = o

    return pl.pallas_call(
        body,
        grid=(GRID_I,),
        in_specs=[
            pl.BlockSpec((U_BLK, 4 * PK), lambda i: (i, 0)),
            pl.BlockSpec((PK, HIDDEN), lambda i: (0, 0)),
            pl.BlockSpec((PK, HIDDEN), lambda i: (0, 0)),
            pl.BlockSpec((1, HIDDEN), lambda i: (0, 0)),
            pl.BlockSpec((HIDDEN, HIDDEN), lambda i: (0, 0)),
            pl.BlockSpec((1, HIDDEN), lambda i: (0, 0)),
        ],
        out_specs=pl.BlockSpec((4, 1, U_BLK, HIDDEN),
                               lambda i: (0, i, 0, 0)),
        out_shape=jax.ShapeDtypeStruct((4, GRID_I, U_BLK, HIDDEN),
                                       jnp.float32),
    )(agg_u32, W1e, W1o, b1, W2, b2)


def kernel(kg_spatial_matrix, rel_table, W1, b1, W2, b2):
    table_bf = rel_table.at[0].set(0.0).astype(jnp.bfloat16)
    table_u32 = jax.lax.bitcast_convert_type(
        table_bf.reshape(NUM_RELATIONS, PK, 2), jnp.uint32
    )
    agg_u32 = _sc_gather_mean(kg_spatial_matrix, table_u32)
    out4 = _mlp(agg_u32, W1[0::2], W1[1::2], b1.reshape(1, HIDDEN), W2,
                b2.reshape(1, HIDDEN))
    return out4.reshape(S, B, HIDDEN).transpose(1, 0, 2)


# speedup vs baseline: 1.1123x; 1.0914x over previous
import functools

import jax
import jax.numpy as jnp
from jax import lax
from jax.experimental import pallas as pl
from jax.experimental.pallas import tpu as pltpu
from jax.experimental.pallas import tpu_sc as plsc

NUM_RELATIONS = 1000
EMBED_DIM = 64
HIDDEN = 128
B, S = 1024, 20
N = B * S
NC, NS = 2, 16
NW = NC * NS
PK = EMBED_DIM // 2
QROWS = N // 4
Q_PER_W = QROWS // NW
Q_CHUNK = 2
N_CHUNKS = Q_PER_W // Q_CHUNK
IDX_PER_K = Q_CHUNK * S
NBUF = 2


def _sc_gather_mean(idx_flat, table_u32):
    mesh = plsc.VectorSubcoreMesh(core_axis_name="c", subcore_axis_name="s")

    @functools.partial(
        pl.kernel,
        out_type=jax.ShapeDtypeStruct((QROWS, 4 * PK), jnp.uint32),
        mesh=mesh,
        scratch_types=[
            pltpu.VMEM((4, Q_PER_W * S), jnp.int32),
            [pltpu.VMEM((4 * IDX_PER_K, PK), jnp.uint32)] * NBUF,
            [pltpu.VMEM((Q_CHUNK, 4 * PK), jnp.uint32)] * NBUF,
            [pltpu.SemaphoreType.DMA] * NBUF,
            [pltpu.SemaphoreType.DMA] * NBUF,
            pltpu.VMEM_SHARED((NUM_RELATIONS, PK), jnp.uint32),
        ],
        compiler_params=pltpu.CompilerParams(use_tc_tiling_on_sc=False),
    )
    def k(idx_hbm, table_hbm, agg_hbm, idx_v, rows_v, out_v, gsem, osem,
          tab_sh):
        wid = lax.axis_index("s") * NC + lax.axis_index("c")
        q0 = wid * Q_PER_W

        @pl.when(lax.axis_index("s") == 0)
        def _():
            pltpu.sync_copy(table_hbm, tab_sh)

        for kk in range(4):
            pltpu.sync_copy(
                idx_hbm.at[pl.ds(kk * QROWS * S + q0 * S, Q_PER_W * S)],
                idx_v.at[kk],
            )
        plsc.subcore_barrier()

        def gather(t, b):
            for kk in range(4):
                pltpu.make_async_copy(
                    tab_sh.at[idx_v.at[kk, pl.ds(t * IDX_PER_K, IDX_PER_K)]],
                    rows_v[b].at[pl.ds(kk * IDX_PER_K, IDX_PER_K)],
                    gsem[b],
                ).start()

        def gather_wait(b):
            for kk in range(4):
                pltpu.make_async_copy(
                    tab_sh.at[idx_v.at[0, pl.ds(0, IDX_PER_K)]],
                    rows_v[b].at[pl.ds(kk * IDX_PER_K, IDX_PER_K)],
                    gsem[b],
                ).wait()

        def out_start(t, b):
            pltpu.make_async_copy(
                out_v[b], agg_hbm.at[pl.ds(q0 + t * Q_CHUNK, Q_CHUNK)],
                osem[b],
            ).start()

        def out_wait(b):
            pltpu.make_async_copy(
                out_v[b], agg_hbm.at[pl.ds(0, Q_CHUNK)], osem[b]
            ).wait()

        for b in range(NBUF):
            gather(b, b)

        def outer(tt, carry):
            for b in range(NBUF):
                t = tt * NBUF + b
                gather_wait(b)
                @pl.when(t >= NBUF)
                def _():
                    out_wait(b)

                rv, ov = rows_v[b], out_v[b]
                hi_mask = jnp.uint32(0xFFFF0000)
                for p in range(Q_CHUNK):
                    for kk in range(4):
                        for c in range(PK // 16):
                            acc_a = jnp.zeros((16,), jnp.float32)
                            acc_b = jnp.zeros((16,), jnp.float32)
                            for j in range(S):
                                u = rv[kk * IDX_PER_K + p * S + j,
                                       pl.ds(c * 16, 16)]
                                a = lax.bitcast_convert_type(
                                    u << 16, jnp.float32)
                                bb = lax.bitcast_convert_type(
                                    u & hi_mask, jnp.float32)
                                acc_a = acc_a + a
                                acc_b = acc_b + bb
                            ua = lax.bitcast_convert_type(
                                acc_a * (1.0 / S), jnp.uint32)
                            ub = lax.bitcast_convert_type(
                                acc_b * (1.0 / S), jnp.uint32)
                            ov[p, pl.ds(kk * PK + c * 16, 16)] = (
                                (ua >> 16) | (ub & hi_mask)
                            )
                out_start(t, b)

                @pl.when(t + NBUF < N_CHUNKS)
                def _():
                    gather(t + NBUF, b)

            return carry

        lax.fori_loop(0, N_CHUNKS // NBUF, outer, None)
        for b in range(NBUF):
            out_wait(b)

    return k(idx_flat, table_u32)


GRID_I = 4
U_BLK = QROWS // GRID_I
BATCH_BLK = U_BLK // S


def _mlp(agg_u32, W1e, W1o, b1, W2, b2):
    def body(a_ref, w1e_ref, w1o_ref, b1_ref, w2_ref, b2_ref, o_ref):
        u = a_ref[...]
        a_even = lax.bitcast_convert_type(u << 16, jnp.float32)
        a_odd = lax.bitcast_convert_type(u & jnp.uint32(0xFFFF0000),
                                         jnp.float32)
        for kk in range(4):
            ek = a_even[:, kk * PK:(kk + 1) * PK]
            ok = a_odd[:, kk * PK:(kk + 1) * PK]
            h = (
                jnp.dot(ek, w1e_ref[...], preferred_element_type=jnp.float32)
                + jnp.dot(ok, w1o_ref[...],
                          preferred_element_type=jnp.float32)
            )
            h = jnp.maximum(h + b1_ref[...], 0.0)
            o = (
                jnp.dot(h, w2_ref[...], preferred_element_type=jnp.float32)
                + b2_ref[...]
            )
            o_ref[kk] = o.reshape(BATCH_BLK, S, HIDDEN)

    return pl.pallas_call(
        body,
        grid=(GRID_I,),
        in_specs=[
            pl.BlockSpec((U_BLK, 4 * PK), lambda i: (i, 0)),
            pl.BlockSpec((PK, HIDDEN), lambda i: (0, 0)),
            pl.BlockSpec((PK, HIDDEN), lambda i: (0, 0)),
            pl.BlockSpec((1, HIDDEN), lambda i: (0, 0)),
            pl.BlockSpec((HIDDEN, HIDDEN), lambda i: (0, 0)),
            pl.BlockSpec((1, HIDDEN), lambda i: (0, 0)),
        ],
        out_specs=pl.BlockSpec((4, BATCH_BLK, S, HIDDEN),
                               lambda i: (0, i, 0, 0)),
        out_shape=jax.ShapeDtypeStruct((4, B // 4, S, HIDDEN), jnp.float32),
    )(agg_u32, W1e, W1o, b1, W2, b2)


def kernel(kg_spatial_matrix, rel_table, W1, b1, W2, b2):
    table_bf = rel_table.at[0].set(0.0).astype(jnp.bfloat16)
    table_u32 = jax.lax.bitcast_convert_type(
        table_bf.reshape(NUM_RELATIONS, PK, 2), jnp.uint32
    )
    idx_flat = kg_spatial_matrix.reshape(-1)
    agg_u32 = _sc_gather_mean(idx_flat, table_u32)
    out4 = _mlp(agg_u32, W1[0::2], W1[1::2], b1.reshape(1, HIDDEN), W2,
                b2.reshape(1, HIDDEN))
    return out4.reshape(B, S, HIDDEN)
